# Initial kernel scaffold; baseline (speedup 1.0000x reference)
import jax
import jax.numpy as jnp
from jax.experimental import pallas as pl


def kernel(hidden_states, Wg, W1, W2, W3, Ws1, Ws2, Ws3):
    raise NotImplementedError("write your pallas kernel here")


# speedup vs baseline: 1.1724x; 1.1724x over previous
import functools

import jax
import jax.numpy as jnp
from jax.experimental import pallas as pl
from jax.experimental.pallas import tpu as pltpu

E = 64
K = 6
H = 128
M = 80
MS = 160
N = 2048


def _silu(x):
    return x * jax.nn.sigmoid(x)


def _moe_body(x_ref, wg_ref, w1_ref, w2_ref, w3_ref, ws1_ref, ws2_ref,
              ws3_ref, y_ref, comb_ref):
    e = pl.program_id(0)
    x = x_ref[...]

    @pl.when(e == 0)
    def _init():
        logits = jax.lax.dot_general(
            x, wg_ref[...], (((1,), (1,)), ((), ())),
            preferred_element_type=jnp.float32)
        mx = jnp.max(logits, axis=-1, keepdims=True)
        p = jnp.exp(logits - mx)
        scores = p / jnp.sum(p, axis=-1, keepdims=True)
        lane = jax.lax.broadcasted_iota(jnp.int32, scores.shape, 1)
        work = scores
        sel = jnp.zeros_like(scores)
        total = jnp.zeros((N, 1), jnp.float32)
        for _ in range(K):
            mval = jnp.max(work, axis=-1, keepdims=True)
            ismax = work == mval
            first = jnp.min(jnp.where(ismax, lane, E), axis=-1,
                            keepdims=True)
            pick = lane == first
            sel = sel + jnp.where(pick, work, 0.0)
            total = total + mval
            work = jnp.where(pick, -1.0, work)
        comb_ref[...] = sel / (total + 1e-20)
        g = jnp.dot(x, ws1_ref[...], preferred_element_type=jnp.float32)
        u = jnp.dot(x, ws2_ref[...], preferred_element_type=jnp.float32)
        y_ref[...] = jnp.dot(_silu(g) * u, ws3_ref[...],
                             preferred_element_type=jnp.float32)

    g = jnp.dot(x, w1_ref[0], preferred_element_type=jnp.float32)
    u = jnp.dot(x, w2_ref[0], preferred_element_type=jnp.float32)
    o = jnp.dot(_silu(g) * u, w3_ref[0], preferred_element_type=jnp.float32)
    lane = jax.lax.broadcasted_iota(jnp.int32, (N, E), 1)
    w_e = jnp.sum(jnp.where(lane == e, comb_ref[...], 0.0), axis=-1,
                  keepdims=True)
    y_ref[...] += o * w_e


@functools.partial(jax.jit, static_argnames=())
def kernel(hidden_states, Wg, W1, W2, W3, Ws1, Ws2, Ws3):
    B, S, h = hidden_states.shape
    x = hidden_states.reshape(N, H)
    y = pl.pallas_call(
        _moe_body,
        grid=(E,),
        in_specs=[
            pl.BlockSpec((N, H), lambda e: (0, 0)),
            pl.BlockSpec((E, H), lambda e: (0, 0)),
            pl.BlockSpec((1, H, M), lambda e: (e, 0, 0)),
            pl.BlockSpec((1, H, M), lambda e: (e, 0, 0)),
            pl.BlockSpec((1, M, H), lambda e: (e, 0, 0)),
            pl.BlockSpec((H, MS), lambda e: (0, 0)),
            pl.BlockSpec((H, MS), lambda e: (0, 0)),
            pl.BlockSpec((MS, H), lambda e: (0, 0)),
        ],
        out_specs=pl.BlockSpec((N, H), lambda e: (0, 0)),
        out_shape=jax.ShapeDtypeStruct((N, H), jnp.float32),
        scratch_shapes=[pltpu.VMEM((N, E), jnp.float32)],
        compiler_params=pltpu.CompilerParams(
            dimension_semantics=("arbitrary",)),
    )(x, Wg, W1, W2, W3, Ws1, Ws2, Ws3)
    return y.reshape(B, S, h)
